# baseline (speedup 1.0000x reference)
import numpy as np
import jax
import jax.numpy as jnp
from jax import lax
from jax.experimental import pallas as pl
from jax.experimental.pallas import tpu as pltpu
from jax.experimental.pallas import tpu_sc as plsc

VOCAB = 30522
D = 768
MAX_LEN = 512
NSEG = 3
B = 64
S = 512

NC = 2
NS = 16
NW = NC * NS
L = 16
SPW = S // NW
G = 1
GT = G * SPW
GN = B // G
NSLOT = 4
LOOKAHEAD = 2


def _positional_encoding_np(max_len, d):
    position = np.arange(max_len, dtype=np.float32)[:, None]
    div_term = np.exp(np.arange(0, d, 2, dtype=np.float32) * -(np.log(10000.0) / d))
    pe = np.zeros((max_len, d), dtype=np.float32)
    pe[:, 0::2] = np.sin(position * div_term)
    pe[:, 1::2] = np.cos(position * div_term)
    return pe


_PE = _positional_encoding_np(MAX_LEN, D)


def _bert_embed_body(x_hbm, lbl_hbm, tok_hbm, seg_hbm, pe_hbm, out_hbm,
                     idx_v, lbl_v, pe_v, seg_v, posseg_v, rows_v,
                     *sems):
    g_sems = sems[:NSLOT]
    g2_sems = sems[NSLOT:NSLOT * 2]
    o_sems = sems[NSLOT * 2:NSLOT * 3]

    wid = lax.axis_index("s") * NC + lax.axis_index("c")
    s0 = wid * SPW

    H = GT // 2

    def start_gather(p, slot):
        pltpu.async_copy(tok_hbm.at[idx_v.at[pl.ds(p * GT, H)]],
                         rows_v.at[slot, pl.ds(0, H)], g_sems[slot])
        pltpu.async_copy(tok_hbm.at[idx_v.at[pl.ds(p * GT + H, H)]],
                         rows_v.at[slot, pl.ds(H, H)], g2_sems[slot])

    def wait_gather(p, slot):
        pltpu.make_async_copy(tok_hbm.at[idx_v.at[pl.ds(p * GT, H)]],
                              rows_v.at[slot, pl.ds(0, H)], g_sems[slot]).wait()
        pltpu.make_async_copy(tok_hbm.at[idx_v.at[pl.ds(p * GT + H, H)]],
                              rows_v.at[slot, pl.ds(H, H)], g2_sems[slot]).wait()

    def start_out(p, slot):
        for g in range(G):
            pltpu.async_copy(rows_v.at[slot, pl.ds(g * SPW, SPW)],
                             out_hbm.at[G * p + g, pl.ds(s0, SPW), :],
                             o_sems[slot])

    def wait_out(p, slot):
        for g in range(G):
            pltpu.make_async_copy(rows_v.at[slot, pl.ds(g * SPW, SPW)],
                                  out_hbm.at[G * p + g, pl.ds(s0, SPW), :],
                                  o_sems[slot]).wait()


    pltpu.sync_copy(x_hbm.at[pl.ds(wid * (B * SPW), B * SPW)], idx_v)
    lbl_cp = pltpu.async_copy(lbl_hbm.at[pl.ds(wid * (B * SPW), B * SPW)],
                              lbl_v.at[pl.ds(0, B * SPW)], o_sems[0])
    seg_cp = pltpu.async_copy(seg_hbm, seg_v, o_sems[1])
    pe_cp = pltpu.async_copy(pe_hbm.at[pl.ds(s0, SPW), :], pe_v, o_sems[2])
    for i in range(LOOKAHEAD):
        start_gather(i, i)
    lbl_cp.wait()
    seg_cp.wait()
    pe_cp.wait()

    for lbl in range(NSEG):
        def _pp(j, _, lbl=lbl):
            row = lbl * SPW + j

            @plsc.parallel_loop(0, D, L, unroll=8)
            def _pk(c):
                sl = pl.ds(c, L)
                posseg_v[row, sl] = pe_v[j, sl] + seg_v[lbl, sl]
            return 0
        lax.fori_loop(0, SPW, _pp, 0)

    def compute(p, slot):
        def _j(j, _):
            lbl = lbl_v[pl.ds(p * GT + j, L)][0]
            row = lbl * SPW + (j & (SPW - 1))

            @plsc.parallel_loop(0, D, L, unroll=8)
            def _k(c):
                sl = pl.ds(c, L)
                plsc.addupdate(rows_v.at[slot, j, sl], posseg_v[row, sl])
            return 0
        lax.fori_loop(0, GT, _j, 0)

    def step(p, slot, do_wait_out, do_gather):
        gslot = (slot + LOOKAHEAD) % NSLOT
        if do_wait_out:
            wait_out(p - (NSLOT - LOOKAHEAD), gslot)
        if do_gather:
            start_gather(p + LOOKAHEAD, gslot)
        wait_gather(p, slot)
        compute(p, slot)
        start_out(p, slot)

    FRONT = NSLOT - LOOKAHEAD
    for p in range(FRONT):
        step(p, p % NSLOT, False, True)

    STEADY = ((GN - LOOKAHEAD - FRONT) // NSLOT) * NSLOT

    def pipe(t, _):
        p = NSLOT * t + FRONT
        for i in range(NSLOT):
            step(p + i, (FRONT + i) % NSLOT, True, True)
        return 0

    lax.fori_loop(0, STEADY // NSLOT, pipe, 0)

    for p in range(FRONT + STEADY, GN):
        step(p, p % NSLOT, True, p + LOOKAHEAD < GN)
    for p in range(GN - (NSLOT - LOOKAHEAD), GN):
        wait_out(p, p % NSLOT)


@jax.jit
def _bert_embed(x, segment_label, token_table, segment_table, pe):
    mesh = plsc.VectorSubcoreMesh(core_axis_name="c", subcore_axis_name="s",
                                  num_cores=NC, num_subcores=NS)
    scratch = [
        pltpu.VMEM((B * SPW,), jnp.int32),
        pltpu.VMEM((B * SPW + L,), jnp.int32),
        pltpu.VMEM((SPW, D), jnp.float32),
        pltpu.VMEM((8, D), jnp.float32),
        pltpu.VMEM((NSEG * SPW, D), jnp.float32),
        pltpu.VMEM((NSLOT, GT, D), jnp.float32),
    ] + [pltpu.SemaphoreType.DMA] * (3 * NSLOT)
    f = pl.kernel(
        _bert_embed_body,
        out_type=jax.ShapeDtypeStruct((B, S, D), jnp.float32),
        mesh=mesh,
        scratch_types=scratch,
    )

    def _worker_major(a):
        return a.reshape(B, NW, SPW).transpose(1, 0, 2).reshape(NW * B * SPW)

    seg_pad = jnp.zeros((8, D), jnp.float32).at[:NSEG].set(segment_table)
    return f(_worker_major(x), _worker_major(segment_label),
             token_table, seg_pad, pe)


def kernel(x, segment_label, token_table, segment_table):
    pe = jnp.asarray(_PE)
    return _bert_embed(x, segment_label, token_table, segment_table, pe)


# speedup vs baseline: 1.0065x; 1.0065x over previous
import numpy as np
import jax
import jax.numpy as jnp
from jax import lax
from jax.experimental import pallas as pl
from jax.experimental.pallas import tpu as pltpu
from jax.experimental.pallas import tpu_sc as plsc

VOCAB = 30522
D = 768
MAX_LEN = 512
NSEG = 3
B = 64
S = 512

NC = 2
NS = 16
NW = NC * NS
L = 16
SPW = S // NW
G = 1
GT = G * SPW
GN = B // G
NSLOT = 4
LOOKAHEAD = 2


def _positional_encoding_np(max_len, d):
    position = np.arange(max_len, dtype=np.float32)[:, None]
    div_term = np.exp(np.arange(0, d, 2, dtype=np.float32) * -(np.log(10000.0) / d))
    pe = np.zeros((max_len, d), dtype=np.float32)
    pe[:, 0::2] = np.sin(position * div_term)
    pe[:, 1::2] = np.cos(position * div_term)
    return pe


_PE = _positional_encoding_np(MAX_LEN, D)


def _bert_embed_body(x_hbm, lbl_hbm, tok_hbm, seg_hbm, pe_hbm, out_hbm,
                     idx_v, lbl_v, pe_v, seg_v, posseg_v, rows_v,
                     *sems):
    g_sems = sems[:NSLOT]
    o_sems = sems[NSLOT:NSLOT * 2]

    wid = lax.axis_index("s") * NC + lax.axis_index("c")
    s0 = wid * SPW

    def start_gather(p, slot):
        pltpu.async_copy(tok_hbm.at[idx_v.at[pl.ds(p * GT, GT)]],
                         rows_v.at[slot], g_sems[slot])

    def wait_gather(p, slot):
        pltpu.make_async_copy(tok_hbm.at[idx_v.at[pl.ds(p * GT, GT)]],
                              rows_v.at[slot], g_sems[slot]).wait()

    def start_out(p, slot):
        for g in range(G):
            pltpu.async_copy(rows_v.at[slot, pl.ds(g * SPW, SPW)],
                             out_hbm.at[G * p + g, pl.ds(s0, SPW), :],
                             o_sems[slot])

    def wait_out(p, slot):
        for g in range(G):
            pltpu.make_async_copy(rows_v.at[slot, pl.ds(g * SPW, SPW)],
                                  out_hbm.at[G * p + g, pl.ds(s0, SPW), :],
                                  o_sems[slot]).wait()


    pltpu.sync_copy(x_hbm.at[pl.ds(wid * (B * SPW), B * SPW)], idx_v)
    lbl_cp = pltpu.async_copy(lbl_hbm.at[pl.ds(wid * (B * SPW), B * SPW)],
                              lbl_v.at[pl.ds(0, B * SPW)], o_sems[0])
    seg_cp = pltpu.async_copy(seg_hbm, seg_v, o_sems[1])
    pe_cp = pltpu.async_copy(pe_hbm.at[pl.ds(s0, SPW), :], pe_v, o_sems[2])
    for i in range(LOOKAHEAD):
        start_gather(i, i)
    lbl_cp.wait()
    seg_cp.wait()
    pe_cp.wait()

    for lbl in range(NSEG):
        def _pp(j, _, lbl=lbl):
            row = lbl * SPW + j

            @plsc.parallel_loop(0, D, L, unroll=8)
            def _pk(c):
                sl = pl.ds(c, L)
                posseg_v[row, sl] = pe_v[j, sl] + seg_v[lbl, sl]
            return 0
        lax.fori_loop(0, SPW, _pp, 0)

    def compute(p, slot):
        def _j(j, _):
            lbl = lbl_v[pl.ds(p * GT + j, L)][0]
            row = lbl * SPW + (j & (SPW - 1))

            @plsc.parallel_loop(0, D, L, unroll=8)
            def _k(c):
                sl = pl.ds(c, L)
                plsc.addupdate(rows_v.at[slot, j, sl], posseg_v[row, sl])
            return 0
        lax.fori_loop(0, GT, _j, 0)

    def step(p, slot, do_wait_out, do_gather):
        gslot = (slot + LOOKAHEAD) % NSLOT
        if do_wait_out:
            wait_out(p - (NSLOT - LOOKAHEAD), gslot)
        if do_gather:
            start_gather(p + LOOKAHEAD, gslot)
        wait_gather(p, slot)
        compute(p, slot)
        start_out(p, slot)

    FRONT = NSLOT - LOOKAHEAD
    for p in range(FRONT):
        step(p, p % NSLOT, False, True)

    STEADY = ((GN - LOOKAHEAD - FRONT) // NSLOT) * NSLOT

    def pipe(t, _):
        p = NSLOT * t + FRONT
        for i in range(NSLOT):
            step(p + i, (FRONT + i) % NSLOT, True, True)
        return 0

    lax.fori_loop(0, STEADY // NSLOT, pipe, 0)

    for p in range(FRONT + STEADY, GN):
        step(p, p % NSLOT, True, p + LOOKAHEAD < GN)
    for p in range(GN - (NSLOT - LOOKAHEAD), GN):
        wait_out(p, p % NSLOT)


@jax.jit
def _bert_embed(x, segment_label, token_table, segment_table, pe):
    mesh = plsc.VectorSubcoreMesh(core_axis_name="c", subcore_axis_name="s",
                                  num_cores=NC, num_subcores=NS)
    scratch = [
        pltpu.VMEM((B * SPW,), jnp.int32),
        pltpu.VMEM((B * SPW + L,), jnp.int32),
        pltpu.VMEM((SPW, D), jnp.float32),
        pltpu.VMEM((8, D), jnp.float32),
        pltpu.VMEM((NSEG * SPW, D), jnp.float32),
        pltpu.VMEM((NSLOT, GT, D), jnp.float32),
    ] + [pltpu.SemaphoreType.DMA] * (2 * NSLOT)
    f = pl.kernel(
        _bert_embed_body,
        out_type=jax.ShapeDtypeStruct((B, S, D), jnp.float32),
        mesh=mesh,
        scratch_types=scratch,
    )

    def _worker_major(a):
        return a.reshape(B, NW, SPW).transpose(1, 0, 2).reshape(NW * B * SPW)

    seg_pad = jnp.zeros((8, D), jnp.float32).at[:NSEG].set(segment_table)
    return f(_worker_major(x), _worker_major(segment_label),
             token_table, seg_pad, pe)


def kernel(x, segment_label, token_table, segment_table):
    pe = jnp.asarray(_PE)
    return _bert_embed(x, segment_label, token_table, segment_table, pe)
